# baseline (speedup 1.0000x reference)
import functools
import jax
import jax.numpy as jnp
from jax import lax
from jax.experimental import pallas as pl
from jax.experimental.pallas import tpu as pltpu
from jax.experimental.pallas import tpu_sc as plsc

A = 128
NBR = 16
EPS = 1e-5


_MASK_HI = -65536


def _unpack(word):
    lo = lax.bitcast_convert_type(jnp.left_shift(word, 16), jnp.float32)
    hi = lax.bitcast_convert_type(jnp.bitwise_and(word, _MASK_HI), jnp.float32)
    return lo, hi


def _proj_body(atom_ref, ws_ref, wn_ref, b_ref, p_ref, t_ref):
    x = atom_ref[...]
    p_ref[...] = jnp.dot(x, ws_ref[...], preferred_element_type=jnp.float32) + b_ref[...]
    t = jnp.dot(x, wn_ref[...], preferred_element_type=jnp.float32)
    lo = lax.bitcast_convert_type(t[:, :A], jnp.int32) + 0x8000
    hi = lax.bitcast_convert_type(t[:, A:], jnp.int32) + 0x8000
    t_ref[...] = jnp.bitwise_or(
        jnp.bitwise_and(hi, _MASK_HI), lax.shift_right_logical(lo, 16))


def _projections(atom, ws, wn, b2d, tile):
    n = atom.shape[0]
    grid = (n // tile,)
    return pl.pallas_call(
        _proj_body,
        grid=grid,
        in_specs=[
            pl.BlockSpec((tile, A), lambda i: (i, 0)),
            pl.BlockSpec((A, 2 * A), lambda i: (0, 0)),
            pl.BlockSpec((A, 2 * A), lambda i: (0, 0)),
            pl.BlockSpec((1, 2 * A), lambda i: (0, 0)),
        ],
        out_specs=[
            pl.BlockSpec((tile, 2 * A), lambda i: (i, 0)),
            pl.BlockSpec((tile, A), lambda i: (i, 0)),
        ],
        out_shape=[
            jax.ShapeDtypeStruct((n, 2 * A), jnp.float32),
            jax.ShapeDtypeStruct((n, A), jnp.int32),
        ],
    )(atom, ws, wn, b2d)


def _make_sc_gather(nrows, d, chunk, dtype):
    info = plsc.get_sparse_core_info()
    nw = info.num_cores * info.num_subcores
    per_w = nrows // nw
    nch = per_w // chunk
    mesh = plsc.VectorSubcoreMesh(core_axis_name="c", subcore_axis_name="s")

    nbuf = 3
    assert nch >= nbuf

    @functools.partial(
        pl.kernel,
        mesh=mesh,
        out_type=jax.ShapeDtypeStruct((nrows, d), dtype),
        scratch_types=[
            pltpu.VMEM((per_w,), jnp.int32),
        ] + [pltpu.VMEM((chunk, d), dtype)] * nbuf
          + [pltpu.SemaphoreType.DMA] * (2 * nbuf),
    )
    def gk(t_hbm, idx_hbm, out_hbm, idx_all, *refs):
        bufs = refs[:nbuf]
        sgs = refs[nbuf:2 * nbuf]
        sws = refs[2 * nbuf:]
        wid = lax.axis_index("s") * info.num_cores + lax.axis_index("c")
        base = wid * per_w
        pltpu.sync_copy(idx_hbm.at[pl.ds(base, per_w)], idx_all)

        def g_copy(k, b):
            return pltpu.make_async_copy(
                t_hbm.at[idx_all.at[pl.ds(k * chunk, chunk)]], bufs[b], sgs[b])

        def w_copy(k, b):
            return pltpu.make_async_copy(
                bufs[b], out_hbm.at[pl.ds(base + k * chunk, chunk)], sws[b])

        def iter_ops(k, b):
            @pl.when(k >= nbuf)
            def _():
                w_copy(k - nbuf, b).wait()

            g_copy(k, b).start()

            @pl.when(k >= 2)
            def _():
                bd = (b - 2) % nbuf
                g_copy(k - 2, bd).wait()
                w_copy(k - 2, bd).start()

        def body(o, carry):
            for b in range(nbuf):
                iter_ops(o * nbuf + b, b)
            return carry

        ngroups = nch // nbuf
        lax.fori_loop(0, ngroups, body, 0)
        for k in range(ngroups * nbuf, nch):
            iter_ops(k, k % nbuf)
        for k in (nch - 2, nch - 1):
            b = k % nbuf
            g_copy(k, b).wait()
            w_copy(k, b).start()
        for k in (nch - 3, nch - 2, nch - 1):
            w_copy(k, k % nbuf).wait()

    return gk


def _edge_halves(g_ref, nbr_ref, p_ref, we_ref):
    tn, m_, a = g_ref.shape
    tf, tc = _unpack(g_ref[...].reshape(tn * m_, a))
    ep = jnp.dot(nbr_ref[...].reshape(tn * m_, NBR), we_ref[...],
                 preferred_element_type=jnp.float32)
    p = p_ref[...]
    pf = jnp.broadcast_to(p[:, None, :A], (tn, m_, a)).reshape(tn * m_, a)
    pc = jnp.broadcast_to(p[:, None, A:], (tn, m_, a)).reshape(tn * m_, a)
    return tf + pf + ep[:, :A], tc + pc + ep[:, A:]


def _fused_body(g_ref, nbr_ref, p_ref, we_ref, g1_ref, b1_ref,
                sum_ref, sq_ref, s_ref, ssum_ref, ssq_ref, *, cnt):
    ph = pl.program_id(0)
    i = pl.program_id(1)
    tn, m_, a = g_ref.shape
    vf, vc = _edge_halves(g_ref, nbr_ref, p_ref, we_ref)

    @pl.when(jnp.logical_and(ph == 0, i == 0))
    def _():
        sum_ref[...] = jnp.zeros_like(sum_ref)
        sq_ref[...] = jnp.zeros_like(sq_ref)

    @pl.when(ph == 0)
    def _():
        sum_ref[...] += jnp.concatenate(
            [jnp.sum(vf, axis=0, keepdims=True),
             jnp.sum(vc, axis=0, keepdims=True)], axis=1)
        sq_ref[...] += jnp.concatenate(
            [jnp.sum(vf * vf, axis=0, keepdims=True),
             jnp.sum(vc * vc, axis=0, keepdims=True)], axis=1)

    @pl.when(jnp.logical_and(ph == 1, i == 0))
    def _():
        ssum_ref[...] = jnp.zeros_like(ssum_ref)
        ssq_ref[...] = jnp.zeros_like(ssq_ref)

    @pl.when(ph == 1)
    def _():
        mu = sum_ref[...] * (1.0 / cnt)
        var = sq_ref[...] * (1.0 / cnt) - mu * mu
        sc = g1_ref[...] * jax.lax.rsqrt(var + EPS)
        sh = b1_ref[...] - mu * sc
        uf = vf * sc[:, :A] + sh[:, :A]
        uc = vc * sc[:, A:] + sh[:, A:]
        prod = (0.5 + 0.5 * jnp.tanh(0.5 * uf)) * jnp.maximum(uc, 0.0)
        s = jnp.sum(prod.reshape(tn, m_, a), axis=1)
        s_ref[...] = s
        ssum_ref[...] += jnp.sum(s, axis=0, keepdims=True)
        ssq_ref[...] += jnp.sum(s * s, axis=0, keepdims=True)


def _fused(g3, nbr, p, we, g1, b1, m, n, tile):
    return pl.pallas_call(
        functools.partial(_fused_body, cnt=float(n * m)),
        grid=(2, n // tile),
        in_specs=[
            pl.BlockSpec((tile, m, A), lambda ph, i: (i, 0, 0)),
            pl.BlockSpec((tile, m, NBR), lambda ph, i: (i, 0, 0)),
            pl.BlockSpec((tile, 2 * A), lambda ph, i: (i, 0)),
            pl.BlockSpec((NBR, 2 * A), lambda ph, i: (0, 0)),
            pl.BlockSpec((1, 2 * A), lambda ph, i: (0, 0)),
            pl.BlockSpec((1, 2 * A), lambda ph, i: (0, 0)),
        ],
        out_specs=[
            pl.BlockSpec((1, 2 * A), lambda ph, i: (0, 0)),
            pl.BlockSpec((1, 2 * A), lambda ph, i: (0, 0)),
            pl.BlockSpec((tile, A), lambda ph, i: (ph * i, 0)),
            pl.BlockSpec((1, A), lambda ph, i: (0, 0)),
            pl.BlockSpec((1, A), lambda ph, i: (0, 0)),
        ],
        out_shape=[
            jax.ShapeDtypeStruct((1, 2 * A), jnp.float32),
            jax.ShapeDtypeStruct((1, 2 * A), jnp.float32),
            jax.ShapeDtypeStruct((n, A), jnp.float32),
            jax.ShapeDtypeStruct((1, A), jnp.float32),
            jax.ShapeDtypeStruct((1, A), jnp.float32),
        ],
    )(g3, nbr, p, we, g1, b1)


def _final_body(atom_ref, s_ref, sc_ref, sh_ref, out_ref):
    out_ref[...] = jnp.maximum(
        atom_ref[...] + s_ref[...] * sc_ref[...] + sh_ref[...], 0.0)


def _final(atom, s, scale2, shift2, tile):
    n = atom.shape[0]
    return pl.pallas_call(
        _final_body,
        grid=(n // tile,),
        in_specs=[
            pl.BlockSpec((tile, A), lambda i: (i, 0)),
            pl.BlockSpec((tile, A), lambda i: (i, 0)),
            pl.BlockSpec((1, A), lambda i: (0, 0)),
            pl.BlockSpec((1, A), lambda i: (0, 0)),
        ],
        out_specs=pl.BlockSpec((tile, A), lambda i: (i, 0)),
        out_shape=jax.ShapeDtypeStruct((n, A), jnp.float32),
    )(atom, s, scale2, shift2)


def kernel(atom_in_fea, nbr_fea, nbr_fea_idx, W_full, b_full,
           bn1_gamma, bn1_beta, bn2_gamma, bn2_beta):
    n, m = nbr_fea_idx.shape
    ws = W_full[:A]
    wn = W_full[A:2 * A]
    we = W_full[2 * A:]
    b2d = b_full.reshape(1, 2 * A)

    p, t = _projections(atom_in_fea, ws, wn, b2d, tile=2000)

    idx_flat = nbr_fea_idx.astype(jnp.int32).reshape(-1)

    g = _make_sc_gather(m * n, A, chunk=200, dtype=jnp.int32)(t, idx_flat)
    g3 = g.reshape(n, m, A)

    vsum, vsq, s, ssum, ssq = _fused(
        g3, nbr_fea, p, we, bn1_gamma.reshape(1, 2 * A),
        bn1_beta.reshape(1, 2 * A), m, n, tile=400)
    mu2 = ssum / float(n)
    var2 = ssq / float(n) - mu2 * mu2
    scale2 = (bn2_gamma / jnp.sqrt(var2 + EPS)).reshape(1, A)
    shift2 = (bn2_beta - mu2 * scale2).reshape(1, A)

    return _final(atom_in_fea, s, scale2, shift2, tile=2000)


# speedup vs baseline: 1.0214x; 1.0214x over previous
import functools
import jax
import jax.numpy as jnp
from jax import lax
from jax.experimental import pallas as pl
from jax.experimental.pallas import tpu as pltpu
from jax.experimental.pallas import tpu_sc as plsc

A = 128
NBR = 16
EPS = 1e-5


_MASK_HI = -65536


def _unpack(word):
    lo = lax.bitcast_convert_type(jnp.left_shift(word, 16), jnp.float32)
    hi = lax.bitcast_convert_type(jnp.bitwise_and(word, _MASK_HI), jnp.float32)
    return lo, hi


def _proj_body(atom_ref, ws_ref, wn_ref, b_ref, p_ref, t_ref):
    x = atom_ref[...]
    p_ref[...] = jnp.dot(x, ws_ref[...], preferred_element_type=jnp.float32) + b_ref[...]
    t = jnp.dot(x, wn_ref[...], preferred_element_type=jnp.float32)
    lo = lax.bitcast_convert_type(t[:, :A], jnp.int32) + 0x8000
    hi = lax.bitcast_convert_type(t[:, A:], jnp.int32) + 0x8000
    t_ref[...] = jnp.bitwise_or(
        jnp.bitwise_and(hi, _MASK_HI), lax.shift_right_logical(lo, 16))


def _projections(atom, ws, wn, b2d, tile):
    n = atom.shape[0]
    grid = (n // tile,)
    return pl.pallas_call(
        _proj_body,
        grid=grid,
        in_specs=[
            pl.BlockSpec((tile, A), lambda i: (i, 0)),
            pl.BlockSpec((A, 2 * A), lambda i: (0, 0)),
            pl.BlockSpec((A, 2 * A), lambda i: (0, 0)),
            pl.BlockSpec((1, 2 * A), lambda i: (0, 0)),
        ],
        out_specs=[
            pl.BlockSpec((tile, 2 * A), lambda i: (i, 0)),
            pl.BlockSpec((tile, A), lambda i: (i, 0)),
        ],
        out_shape=[
            jax.ShapeDtypeStruct((n, 2 * A), jnp.float32),
            jax.ShapeDtypeStruct((n, A), jnp.int32),
        ],
    )(atom, ws, wn, b2d)


def _make_sc_gather(nrows, d, chunk, dtype):
    info = plsc.get_sparse_core_info()
    nw = info.num_cores * info.num_subcores
    per_w = nrows // nw
    nch = per_w // chunk
    mesh = plsc.VectorSubcoreMesh(core_axis_name="c", subcore_axis_name="s")

    nbuf = 3
    assert nch >= nbuf

    @functools.partial(
        pl.kernel,
        mesh=mesh,
        out_type=jax.ShapeDtypeStruct((nrows, d), dtype),
        scratch_types=[
            pltpu.VMEM((per_w,), jnp.int32),
        ] + [pltpu.VMEM((chunk, d), dtype)] * nbuf
          + [pltpu.SemaphoreType.DMA] * (2 * nbuf),
    )
    def gk(t_hbm, idx_hbm, out_hbm, idx_all, *refs):
        bufs = refs[:nbuf]
        sgs = refs[nbuf:2 * nbuf]
        sws = refs[2 * nbuf:]
        wid = lax.axis_index("s") * info.num_cores + lax.axis_index("c")
        base = wid * per_w
        pltpu.sync_copy(idx_hbm.at[pl.ds(base, per_w)], idx_all)

        def g_copy(k, b):
            return pltpu.make_async_copy(
                t_hbm.at[idx_all.at[pl.ds(k * chunk, chunk)]], bufs[b], sgs[b])

        def w_copy(k, b):
            return pltpu.make_async_copy(
                bufs[b], out_hbm.at[pl.ds(base + k * chunk, chunk)], sws[b])

        def iter_ops(k, b):
            @pl.when(k >= nbuf)
            def _():
                w_copy(k - nbuf, b).wait()

            g_copy(k, b).start()

            @pl.when(k >= 2)
            def _():
                bd = (b - 2) % nbuf
                g_copy(k - 2, bd).wait()
                w_copy(k - 2, bd).start()

        def body(o, carry):
            for b in range(nbuf):
                iter_ops(o * nbuf + b, b)
            return carry

        ngroups = nch // nbuf
        lax.fori_loop(0, ngroups, body, 0)
        for k in range(ngroups * nbuf, nch):
            iter_ops(k, k % nbuf)
        for k in (nch - 2, nch - 1):
            b = k % nbuf
            g_copy(k, b).wait()
            w_copy(k, b).start()
        for k in (nch - 3, nch - 2, nch - 1):
            w_copy(k, k % nbuf).wait()

    return gk


def _edge_halves(g_ref, nbr_ref, p_ref, we_ref):
    tn, m_, a = g_ref.shape
    tf, tc = _unpack(g_ref[...].reshape(tn * m_, a))
    ep = jnp.dot(nbr_ref[...].reshape(tn * m_, NBR), we_ref[...],
                 preferred_element_type=jnp.float32)
    p = p_ref[...]
    pf = jnp.broadcast_to(p[:, None, :A], (tn, m_, a)).reshape(tn * m_, a)
    pc = jnp.broadcast_to(p[:, None, A:], (tn, m_, a)).reshape(tn * m_, a)
    return tf + pf + ep[:, :A], tc + pc + ep[:, A:]


def _stats_body(g_ref, nbr_ref, p_ref, we_ref, sum_ref, sq_ref):
    vf, vc = _edge_halves(g_ref, nbr_ref, p_ref, we_ref)

    @pl.when(pl.program_id(0) == 0)
    def _():
        sum_ref[...] = jnp.zeros_like(sum_ref)
        sq_ref[...] = jnp.zeros_like(sq_ref)

    sum_ref[...] += jnp.concatenate(
        [jnp.sum(vf, axis=0, keepdims=True),
         jnp.sum(vc, axis=0, keepdims=True)], axis=1)
    sq_ref[...] += jnp.concatenate(
        [jnp.sum(vf * vf, axis=0, keepdims=True),
         jnp.sum(vc * vc, axis=0, keepdims=True)], axis=1)


def _stats(g3, nbr, p, we, m, n, tile):
    return pl.pallas_call(
        _stats_body,
        grid=(n // tile,),
        in_specs=[
            pl.BlockSpec((tile, m, A), lambda i: (i, 0, 0)),
            pl.BlockSpec((tile, m, NBR), lambda i: (i, 0, 0)),
            pl.BlockSpec((tile, 2 * A), lambda i: (i, 0)),
            pl.BlockSpec((NBR, 2 * A), lambda i: (0, 0)),
        ],
        out_specs=[
            pl.BlockSpec((1, 2 * A), lambda i: (0, 0)),
            pl.BlockSpec((1, 2 * A), lambda i: (0, 0)),
        ],
        out_shape=[
            jax.ShapeDtypeStruct((1, 2 * A), jnp.float32),
            jax.ShapeDtypeStruct((1, 2 * A), jnp.float32),
        ],
    )(g3, nbr, p, we)


def _apply_body(g_ref, nbr_ref, p_ref, we_ref, sc_ref, sh_ref,
                s_ref, ssum_ref, ssq_ref):
    tn, m_, a = g_ref.shape
    tf, tc = _unpack(g_ref[...].reshape(tn * m_, a))
    ep = jnp.dot(nbr_ref[...].reshape(tn * m_, NBR), we_ref[...],
                 preferred_element_type=jnp.float32)
    sc = sc_ref[...]
    p1 = p_ref[...] * sc + sh_ref[...]
    pf = jnp.broadcast_to(p1[:, None, :A], (tn, m_, a)).reshape(tn * m_, a)
    pc = jnp.broadcast_to(p1[:, None, A:], (tn, m_, a)).reshape(tn * m_, a)
    uf = tf * sc[:, :A] + (pf + ep[:, :A])
    uc = tc * sc[:, A:] + (pc + ep[:, A:])
    prod = (0.5 + 0.5 * jnp.tanh(0.5 * uf)) * jnp.maximum(uc, 0.0)
    s = jnp.sum(prod.reshape(tn, m_, a), axis=1)
    s_ref[...] = s

    @pl.when(pl.program_id(0) == 0)
    def _():
        ssum_ref[...] = jnp.zeros_like(ssum_ref)
        ssq_ref[...] = jnp.zeros_like(ssq_ref)

    ssum_ref[...] += jnp.sum(s, axis=0, keepdims=True)
    ssq_ref[...] += jnp.sum(s * s, axis=0, keepdims=True)


def _apply(g3, nbr, p, we, scale, shift, m, n, tile):
    return pl.pallas_call(
        _apply_body,
        grid=(n // tile,),
        in_specs=[
            pl.BlockSpec((tile, m, A), lambda i: (i, 0, 0)),
            pl.BlockSpec((tile, m, NBR), lambda i: (i, 0, 0)),
            pl.BlockSpec((tile, 2 * A), lambda i: (i, 0)),
            pl.BlockSpec((NBR, 2 * A), lambda i: (0, 0)),
            pl.BlockSpec((1, 2 * A), lambda i: (0, 0)),
            pl.BlockSpec((1, 2 * A), lambda i: (0, 0)),
        ],
        out_specs=[
            pl.BlockSpec((tile, A), lambda i: (i, 0)),
            pl.BlockSpec((1, A), lambda i: (0, 0)),
            pl.BlockSpec((1, A), lambda i: (0, 0)),
        ],
        out_shape=[
            jax.ShapeDtypeStruct((n, A), jnp.float32),
            jax.ShapeDtypeStruct((1, A), jnp.float32),
            jax.ShapeDtypeStruct((1, A), jnp.float32),
        ],
    )(g3, nbr, p, we, scale, shift)


def _final_body(atom_ref, s_ref, sc_ref, sh_ref, out_ref):
    out_ref[...] = jnp.maximum(
        atom_ref[...] + s_ref[...] * sc_ref[...] + sh_ref[...], 0.0)


def _final(atom, s, scale2, shift2, tile):
    n = atom.shape[0]
    return pl.pallas_call(
        _final_body,
        grid=(n // tile,),
        in_specs=[
            pl.BlockSpec((tile, A), lambda i: (i, 0)),
            pl.BlockSpec((tile, A), lambda i: (i, 0)),
            pl.BlockSpec((1, A), lambda i: (0, 0)),
            pl.BlockSpec((1, A), lambda i: (0, 0)),
        ],
        out_specs=pl.BlockSpec((tile, A), lambda i: (i, 0)),
        out_shape=jax.ShapeDtypeStruct((n, A), jnp.float32),
    )(atom, s, scale2, shift2)


def kernel(atom_in_fea, nbr_fea, nbr_fea_idx, W_full, b_full,
           bn1_gamma, bn1_beta, bn2_gamma, bn2_beta):
    n, m = nbr_fea_idx.shape
    ws = W_full[:A]
    wn = W_full[A:2 * A]
    we = W_full[2 * A:]
    b2d = b_full.reshape(1, 2 * A)

    p, t = _projections(atom_in_fea, ws, wn, b2d, tile=2000)

    idx_flat = nbr_fea_idx.astype(jnp.int32).reshape(-1)

    g = _make_sc_gather(m * n, A, chunk=200, dtype=jnp.int32)(t, idx_flat)
    g3 = g.reshape(n, m, A)

    vsum, vsq = _stats(g3, nbr_fea, p, we, m, n, tile=400)
    cnt = float(n * m)
    mu = vsum / cnt
    var = vsq / cnt - mu * mu
    scale = (bn1_gamma / jnp.sqrt(var + EPS)).reshape(1, 2 * A)
    shift = (bn1_beta - mu * scale).reshape(1, 2 * A)

    s, ssum, ssq = _apply(
        g3, nbr_fea, p, we * scale, scale, shift, m, n, tile=400)
    mu2 = ssum / float(n)
    var2 = ssq / float(n) - mu2 * mu2
    scale2 = (bn2_gamma / jnp.sqrt(var2 + EPS)).reshape(1, A)
    shift2 = (bn2_beta - mu2 * scale2).reshape(1, A)

    return _final(atom_in_fea, s, scale2, shift2, tile=2000)


# speedup vs baseline: 1.0232x; 1.0018x over previous
import functools
import jax
import jax.numpy as jnp
from jax import lax
from jax.experimental import pallas as pl
from jax.experimental.pallas import tpu as pltpu
from jax.experimental.pallas import tpu_sc as plsc

A = 128
NBR = 16
EPS = 1e-5


_MASK_HI = -65536


def _unpack(word):
    lo = lax.bitcast_convert_type(jnp.left_shift(word, 16), jnp.float32)
    hi = lax.bitcast_convert_type(jnp.bitwise_and(word, _MASK_HI), jnp.float32)
    return lo, hi


def _proj_body(atom_ref, ws_ref, wn_ref, b_ref, p_ref, t_ref):
    x = atom_ref[...]
    p_ref[...] = jnp.dot(x, ws_ref[...], preferred_element_type=jnp.float32) + b_ref[...]
    t = jnp.dot(x, wn_ref[...], preferred_element_type=jnp.float32)
    lo = lax.bitcast_convert_type(t[:, :A], jnp.int32) + 0x8000
    hi = lax.bitcast_convert_type(t[:, A:], jnp.int32) + 0x8000
    t_ref[...] = jnp.bitwise_or(
        jnp.bitwise_and(hi, _MASK_HI), lax.shift_right_logical(lo, 16))


def _projections(atom, ws, wn, b2d, tile):
    n = atom.shape[0]
    grid = (n // tile,)
    return pl.pallas_call(
        _proj_body,
        grid=grid,
        in_specs=[
            pl.BlockSpec((tile, A), lambda i: (i, 0)),
            pl.BlockSpec((A, 2 * A), lambda i: (0, 0)),
            pl.BlockSpec((A, 2 * A), lambda i: (0, 0)),
            pl.BlockSpec((1, 2 * A), lambda i: (0, 0)),
        ],
        out_specs=[
            pl.BlockSpec((tile, 2 * A), lambda i: (i, 0)),
            pl.BlockSpec((tile, A), lambda i: (i, 0)),
        ],
        out_shape=[
            jax.ShapeDtypeStruct((n, 2 * A), jnp.float32),
            jax.ShapeDtypeStruct((n, A), jnp.int32),
        ],
    )(atom, ws, wn, b2d)


def _make_sc_gather(nrows, d, chunk, dtype):
    info = plsc.get_sparse_core_info()
    nw = info.num_cores * info.num_subcores
    per_w = nrows // nw
    nch = per_w // chunk
    mesh = plsc.VectorSubcoreMesh(core_axis_name="c", subcore_axis_name="s")

    nbuf = 4
    assert nch >= nbuf

    @functools.partial(
        pl.kernel,
        mesh=mesh,
        out_type=jax.ShapeDtypeStruct((nrows, d), dtype),
        scratch_types=[
            pltpu.VMEM((per_w,), jnp.int32),
        ] + [pltpu.VMEM((chunk, d), dtype)] * nbuf
          + [pltpu.SemaphoreType.DMA] * (2 * nbuf),
    )
    def gk(t_hbm, idx_hbm, out_hbm, idx_all, *refs):
        bufs = refs[:nbuf]
        sgs = refs[nbuf:2 * nbuf]
        sws = refs[2 * nbuf:]
        wid = lax.axis_index("s") * info.num_cores + lax.axis_index("c")
        base = wid * per_w
        pltpu.sync_copy(idx_hbm.at[pl.ds(base, per_w)], idx_all)

        def g_copy(k, b):
            return pltpu.make_async_copy(
                t_hbm.at[idx_all.at[pl.ds(k * chunk, chunk)]], bufs[b], sgs[b])

        def w_copy(k, b):
            return pltpu.make_async_copy(
                bufs[b], out_hbm.at[pl.ds(base + k * chunk, chunk)], sws[b])

        def iter_ops(k, b):
            @pl.when(k >= nbuf)
            def _():
                w_copy(k - nbuf, b).wait()

            g_copy(k, b).start()

            @pl.when(k >= 2)
            def _():
                bd = (b - 2) % nbuf
                g_copy(k - 2, bd).wait()
                w_copy(k - 2, bd).start()

        def body(o, carry):
            for b in range(nbuf):
                iter_ops(o * nbuf + b, b)
            return carry

        ngroups = nch // nbuf
        lax.fori_loop(0, ngroups, body, 0)
        for k in range(ngroups * nbuf, nch):
            iter_ops(k, k % nbuf)
        for k in (nch - 2, nch - 1):
            b = k % nbuf
            g_copy(k, b).wait()
            w_copy(k, b).start()
        for k in (nch - 3, nch - 2, nch - 1):
            w_copy(k, k % nbuf).wait()

    return gk


def _edge_halves(g_ref, nbr_ref, p_ref, we_ref):
    tn, m_, a = g_ref.shape
    tf, tc = _unpack(g_ref[...].reshape(tn * m_, a))
    ep = jnp.dot(nbr_ref[...].reshape(tn * m_, NBR), we_ref[...],
                 preferred_element_type=jnp.float32)
    p = p_ref[...]
    pf = jnp.broadcast_to(p[:, None, :A], (tn, m_, a)).reshape(tn * m_, a)
    pc = jnp.broadcast_to(p[:, None, A:], (tn, m_, a)).reshape(tn * m_, a)
    return tf + pf + ep[:, :A], tc + pc + ep[:, A:]


def _stats_body(g_ref, nbr_ref, p_ref, we_ref, sum_ref, sq_ref):
    vf, vc = _edge_halves(g_ref, nbr_ref, p_ref, we_ref)

    @pl.when(pl.program_id(0) == 0)
    def _():
        sum_ref[...] = jnp.zeros_like(sum_ref)
        sq_ref[...] = jnp.zeros_like(sq_ref)

    sum_ref[...] += jnp.concatenate(
        [jnp.sum(vf, axis=0, keepdims=True),
         jnp.sum(vc, axis=0, keepdims=True)], axis=1)
    sq_ref[...] += jnp.concatenate(
        [jnp.sum(vf * vf, axis=0, keepdims=True),
         jnp.sum(vc * vc, axis=0, keepdims=True)], axis=1)


def _stats(g3, nbr, p, we, m, n, tile):
    return pl.pallas_call(
        _stats_body,
        grid=(n // tile,),
        in_specs=[
            pl.BlockSpec((tile, m, A), lambda i: (i, 0, 0)),
            pl.BlockSpec((tile, m, NBR), lambda i: (i, 0, 0)),
            pl.BlockSpec((tile, 2 * A), lambda i: (i, 0)),
            pl.BlockSpec((NBR, 2 * A), lambda i: (0, 0)),
        ],
        out_specs=[
            pl.BlockSpec((1, 2 * A), lambda i: (0, 0)),
            pl.BlockSpec((1, 2 * A), lambda i: (0, 0)),
        ],
        out_shape=[
            jax.ShapeDtypeStruct((1, 2 * A), jnp.float32),
            jax.ShapeDtypeStruct((1, 2 * A), jnp.float32),
        ],
    )(g3, nbr, p, we)


def _apply_body(g_ref, nbr_ref, p_ref, we_ref, sc_ref, sh_ref,
                s_ref, ssum_ref, ssq_ref):
    tn, m_, a = g_ref.shape
    tf, tc = _unpack(g_ref[...].reshape(tn * m_, a))
    ep = jnp.dot(nbr_ref[...].reshape(tn * m_, NBR), we_ref[...],
                 preferred_element_type=jnp.float32)
    sc = sc_ref[...]
    p1 = p_ref[...] * sc + sh_ref[...]
    pf = jnp.broadcast_to(p1[:, None, :A], (tn, m_, a)).reshape(tn * m_, a)
    pc = jnp.broadcast_to(p1[:, None, A:], (tn, m_, a)).reshape(tn * m_, a)
    uf = tf * sc[:, :A] + (pf + ep[:, :A])
    uc = tc * sc[:, A:] + (pc + ep[:, A:])
    prod = (0.5 + 0.5 * jnp.tanh(0.5 * uf)) * jnp.maximum(uc, 0.0)
    s = jnp.sum(prod.reshape(tn, m_, a), axis=1)
    s_ref[...] = s

    @pl.when(pl.program_id(0) == 0)
    def _():
        ssum_ref[...] = jnp.zeros_like(ssum_ref)
        ssq_ref[...] = jnp.zeros_like(ssq_ref)

    ssum_ref[...] += jnp.sum(s, axis=0, keepdims=True)
    ssq_ref[...] += jnp.sum(s * s, axis=0, keepdims=True)


def _apply(g3, nbr, p, we, scale, shift, m, n, tile):
    return pl.pallas_call(
        _apply_body,
        grid=(n // tile,),
        in_specs=[
            pl.BlockSpec((tile, m, A), lambda i: (i, 0, 0)),
            pl.BlockSpec((tile, m, NBR), lambda i: (i, 0, 0)),
            pl.BlockSpec((tile, 2 * A), lambda i: (i, 0)),
            pl.BlockSpec((NBR, 2 * A), lambda i: (0, 0)),
            pl.BlockSpec((1, 2 * A), lambda i: (0, 0)),
            pl.BlockSpec((1, 2 * A), lambda i: (0, 0)),
        ],
        out_specs=[
            pl.BlockSpec((tile, A), lambda i: (i, 0)),
            pl.BlockSpec((1, A), lambda i: (0, 0)),
            pl.BlockSpec((1, A), lambda i: (0, 0)),
        ],
        out_shape=[
            jax.ShapeDtypeStruct((n, A), jnp.float32),
            jax.ShapeDtypeStruct((1, A), jnp.float32),
            jax.ShapeDtypeStruct((1, A), jnp.float32),
        ],
    )(g3, nbr, p, we, scale, shift)


def _final_body(atom_ref, s_ref, sc_ref, sh_ref, out_ref):
    out_ref[...] = jnp.maximum(
        atom_ref[...] + s_ref[...] * sc_ref[...] + sh_ref[...], 0.0)


def _final(atom, s, scale2, shift2, tile):
    n = atom.shape[0]
    return pl.pallas_call(
        _final_body,
        grid=(n // tile,),
        in_specs=[
            pl.BlockSpec((tile, A), lambda i: (i, 0)),
            pl.BlockSpec((tile, A), lambda i: (i, 0)),
            pl.BlockSpec((1, A), lambda i: (0, 0)),
            pl.BlockSpec((1, A), lambda i: (0, 0)),
        ],
        out_specs=pl.BlockSpec((tile, A), lambda i: (i, 0)),
        out_shape=jax.ShapeDtypeStruct((n, A), jnp.float32),
    )(atom, s, scale2, shift2)


def kernel(atom_in_fea, nbr_fea, nbr_fea_idx, W_full, b_full,
           bn1_gamma, bn1_beta, bn2_gamma, bn2_beta):
    n, m = nbr_fea_idx.shape
    ws = W_full[:A]
    wn = W_full[A:2 * A]
    we = W_full[2 * A:]
    b2d = b_full.reshape(1, 2 * A)

    p, t = _projections(atom_in_fea, ws, wn, b2d, tile=2000)

    idx_flat = nbr_fea_idx.astype(jnp.int32).reshape(-1)

    g = _make_sc_gather(m * n, A, chunk=200, dtype=jnp.int32)(t, idx_flat)
    g3 = g.reshape(n, m, A)

    vsum, vsq = _stats(g3, nbr_fea, p, we, m, n, tile=400)
    cnt = float(n * m)
    mu = vsum / cnt
    var = vsq / cnt - mu * mu
    scale = (bn1_gamma / jnp.sqrt(var + EPS)).reshape(1, 2 * A)
    shift = (bn1_beta - mu * scale).reshape(1, 2 * A)

    s, ssum, ssq = _apply(
        g3, nbr_fea, p, we * scale, scale, shift, m, n, tile=400)
    mu2 = ssum / float(n)
    var2 = ssq / float(n) - mu2 * mu2
    scale2 = (bn2_gamma / jnp.sqrt(var2 + EPS)).reshape(1, A)
    shift2 = (bn2_beta - mu2 * scale2).reshape(1, A)

    return _final(atom_in_fea, s, scale2, shift2, tile=2000)
